# baseline (speedup 1.0000x reference)
import functools

import jax
import jax.numpy as jnp
from jax import lax
from jax.experimental import pallas as pl
from jax.experimental.pallas import tpu as pltpu
from jax.experimental.pallas import tpu_sc as plsc

VOCAB = 1000000
EMB = 32
BATCH = 4096
SEQ = 200
N = BATCH * SEQ
NW = 32
PER_W = N // NW
CHUNK = 128
G = PER_W // CHUNK
RING = 10
AHEAD = 5
L = 16
VBLK = 4096


def _relayout_block(i_ref, o_ref):
    blk = i_ref[...]
    out = jnp.transpose(blk.reshape(EMB, VBLK // 4, 4), (1, 2, 0))
    o_ref[...] = out.reshape(VBLK // 4, 4 * EMB)


def _make_w_relayout():
    grid = (VOCAB + VBLK - 1) // VBLK
    return pl.pallas_call(
        _relayout_block,
        grid=(grid,),
        in_specs=[pl.BlockSpec((EMB, VBLK), lambda c: (0, c))],
        out_specs=pl.BlockSpec((VBLK // 4, 128), lambda c: (c, 0)),
        out_shape=jax.ShapeDtypeStruct((VOCAB // 4, 4 * EMB), jnp.float32),
    )


def _make_gather_kernel():
    mesh = plsc.VectorSubcoreMesh(core_axis_name="c", subcore_axis_name="s")

    @functools.partial(
        pl.kernel,
        out_type=(
            jax.ShapeDtypeStruct((NW, G, CHUNK, EMB), jnp.float32),
            jax.ShapeDtypeStruct((NW, G, CHUNK), jnp.float32),
        ),
        mesh=mesh,
        compiler_params=pltpu.CompilerParams(use_tc_tiling_on_sc=False),
        scratch_types=(
            [
                pltpu.VMEM((G, CHUNK), jnp.int32),
                pltpu.VMEM((RING, CHUNK, EMB), jnp.float32),
                pltpu.VMEM((G, CHUNK), jnp.float32),
            ]
            + [pltpu.SemaphoreType.DMA] * RING
            + [pltpu.SemaphoreType.DMA] * RING
        ),
    )
    def emb_kernel(x_hbm, w_hbm, out_hbm, mask_hbm, idx_v, rows_v, mask_v,
                   *sems):
        gsems = sems[:RING]
        osems = sems[RING:]
        wid = lax.axis_index("s") * 2 + lax.axis_index("c")

        pltpu.sync_copy(x_hbm.at[wid], idx_v)

        def start_gather(g, r):
            pltpu.async_copy(w_hbm.at[idx_v.at[g]], rows_v.at[r], gsems[r])

        def wait_gather(g, r):
            pltpu.make_async_copy(
                w_hbm.at[idx_v.at[g]], rows_v.at[r], gsems[r]).wait()

        def start_out(g, r):
            pltpu.async_copy(rows_v.at[r], out_hbm.at[wid, g], osems[r])

        def wait_out(g, r):
            pltpu.make_async_copy(
                rows_v.at[r], out_hbm.at[wid, g], osems[r]).wait()

        def compute_mask(g):
            for j in range(CHUNK // L):
                v = idx_v[g, pl.ds(j * L, L)]
                mask_v[g, pl.ds(j * L, L)] = jnp.where(
                    v != 0, jnp.float32(1.0), jnp.float32(0.0))

        def visit(g, r, first_lap):
            compute_mask(g)
            wait_gather(g, r)
            start_out(g, r)
            g2 = g + AHEAD
            r2 = (r + AHEAD) % RING
            if first_lap:
                if g2 < RING:
                    start_gather(g2, r2)
                else:
                    wait_out(g2 - RING, r2)
                    start_gather(g2, r2)
            else:
                @pl.when(g2 < G)
                def _():
                    wait_out(g2 - RING, r2)
                    start_gather(g2, r2)

        for g in range(AHEAD):
            start_gather(g, g)
        for r in range(RING):
            visit(r, r, first_lap=True)

        def step(s, _):
            for r in range(RING):
                visit(s * RING + r, r, first_lap=False)
            return 0

        lax.fori_loop(1, G // RING, step, 0)

        for r in range(RING):
            wait_out(G - RING + r, r)
        pltpu.sync_copy(mask_v, mask_hbm.at[wid])

    return emb_kernel


_w_relayout = None
_emb_kernel = None


def kernel(x, W):
    global _w_relayout, _emb_kernel
    if _w_relayout is None:
        _w_relayout = _make_w_relayout()
        _emb_kernel = _make_gather_kernel()
    w5 = _w_relayout(W.T)
    w_lin = w5.reshape(VOCAB, EMB)
    xf = x.reshape(NW, G, CHUNK).astype(jnp.int32)
    emb, mask = _emb_kernel(xf, w_lin)
    return emb.reshape(BATCH, SEQ, EMB), mask.reshape(BATCH, SEQ)


# speedup vs baseline: 2.9944x; 2.9944x over previous
import functools

import jax
import jax.numpy as jnp
from jax import lax
from jax.experimental import pallas as pl
from jax.experimental.pallas import tpu as pltpu
from jax.experimental.pallas import tpu_sc as plsc

VOCAB = 1000000
EMB = 32
BATCH = 4096
SEQ = 200
N = BATCH * SEQ
NW = 32
PER_W = N // NW
CHUNK = 128
G = PER_W // CHUNK
L = 16

VPAD = 1 << 18
WIDE = 4 * EMB
VBLK = 4096

NRB = 4
AHEAD = 3
NSTG = 2
UNROLL = 4


def _pack_block(i0, i1, i2, i3, o_ref):
    o_ref[...] = jnp.concatenate(
        [jnp.transpose(i0[...], (1, 0)), jnp.transpose(i1[...], (1, 0)),
         jnp.transpose(i2[...], (1, 0)), jnp.transpose(i3[...], (1, 0))],
        axis=1)


def _make_w_pack():
    last_blk = VOCAB // VBLK
    specs = [
        pl.BlockSpec((EMB, VBLK), functools.partial(
            lambda c, k: (0, jnp.minimum((VPAD // VBLK) * k + c, last_blk)),
            k=k))
        for k in range(4)
    ]
    return pl.pallas_call(
        _pack_block,
        grid=(VPAD // VBLK,),
        in_specs=specs,
        out_specs=pl.BlockSpec((VBLK, WIDE), lambda c: (c, 0)),
        out_shape=jax.ShapeDtypeStruct((VPAD, WIDE), jnp.float32),
    )


def _make_gather_kernel():
    mesh = plsc.VectorSubcoreMesh(core_axis_name="c", subcore_axis_name="s")

    @functools.partial(
        pl.kernel,
        out_type=(
            jax.ShapeDtypeStruct((NW, G, CHUNK, EMB), jnp.float32),
            jax.ShapeDtypeStruct((NW, G, CHUNK), jnp.float32),
        ),
        mesh=mesh,
        compiler_params=pltpu.CompilerParams(use_tc_tiling_on_sc=False),
        scratch_types=(
            [
                pltpu.VMEM((G, CHUNK), jnp.int32),
                pltpu.VMEM((NRB, CHUNK), jnp.int32),
                pltpu.VMEM((NRB, CHUNK, WIDE), jnp.float32),
                pltpu.VMEM((NSTG, CHUNK, EMB), jnp.float32),
                pltpu.VMEM((G, CHUNK), jnp.float32),
            ]
            + [pltpu.SemaphoreType.DMA] * NRB
            + [pltpu.SemaphoreType.DMA] * NSTG
        ),
    )
    def emb_kernel(x_hbm, w5_hbm, out_hbm, mask_hbm,
                   idx_v, row_v, wide_v, comp_v, mask_v, *sems):
        gsems = sems[:NRB]
        osems = sems[NRB:]
        wid = lax.axis_index("s") * 2 + lax.axis_index("c")

        pltpu.sync_copy(x_hbm.at[wid], idx_v)

        def prep_gather(g, slot):
            for j in range(CHUNK // L):
                v = idx_v[g, pl.ds(j * L, L)]
                row_v[slot, pl.ds(j * L, L)] = jnp.bitwise_and(v, VPAD - 1)
            pltpu.async_copy(
                w5_hbm.at[row_v.at[slot]], wide_v.at[slot], gsems[slot])

        def wait_gather(slot):
            pltpu.make_async_copy(
                w5_hbm.at[row_v.at[slot]], wide_v.at[slot],
                gsems[slot]).wait()

        def start_out(g, stg):
            pltpu.async_copy(
                comp_v.at[stg], out_hbm.at[wid, g], osems[stg])

        def wait_out(g, stg):
            pltpu.make_async_copy(
                comp_v.at[stg], out_hbm.at[wid, g], osems[stg]).wait()

        def visit(g, u):
            slot = u % NRB
            slot2 = (u + AHEAD) % NRB
            stg = u % NSTG
            g2 = g + AHEAD

            @pl.when(g2 < G)
            def _():
                prep_gather(g2, slot2)

            for j in range(CHUNK // L):
                v = idx_v[g, pl.ds(j * L, L)]
                mask_v[g, pl.ds(j * L, L)] = jnp.where(
                    v != 0, jnp.float32(1.0), jnp.float32(0.0))

            wait_gather(slot)

            @pl.when(g >= NSTG)
            def _():
                wait_out(g - NSTG, stg)

            for j in range(CHUNK // L):
                v = idx_v[g, pl.ds(j * L, L)]
                ext = lax.shift_left(lax.shift_right_logical(v, 18), 5)
                for i in range(L):
                    b = j * L + i
                    off = ext[i]
                    for h in range(2):
                        comp_v[stg, b, pl.ds(h * L, L)] = (
                            wide_v[slot, b, pl.ds(off + h * L, L)])

            start_out(g, stg)

        for g in range(AHEAD):
            prep_gather(g, g % NRB)

        def step(s, _):
            for u in range(UNROLL):
                visit(s * UNROLL + u, u)
            return 0

        lax.fori_loop(0, G // UNROLL, step, 0)

        for u in range(NSTG):
            g = G - NSTG + u
            wait_out(g, g % NSTG)
        pltpu.sync_copy(mask_v, mask_hbm.at[wid])

    return emb_kernel


_w_pack = None
_emb_kernel = None


def kernel(x, W):
    global _w_pack, _emb_kernel
    if _w_pack is None:
        _w_pack = _make_w_pack()
        _emb_kernel = _make_gather_kernel()
    wt = W.T
    w5 = _w_pack(wt, wt, wt, wt)
    xf = x.reshape(NW, G, CHUNK).astype(jnp.int32)
    emb, mask = _emb_kernel(xf, w5)
    return emb.reshape(BATCH, SEQ, EMB), mask.reshape(BATCH, SEQ)


# speedup vs baseline: 3.8418x; 1.2830x over previous
import functools

import jax
import jax.numpy as jnp
from jax import lax
from jax.experimental import pallas as pl
from jax.experimental.pallas import tpu as pltpu
from jax.experimental.pallas import tpu_sc as plsc

VOCAB = 1000000
EMB = 32
BATCH = 4096
SEQ = 200
N = BATCH * SEQ
NW = 32
PER_W = N // NW
CHUNK = 128
G = PER_W // CHUNK
RING = 10
AHEAD = 5
L = 16

VPAD = 1 << 18
WIDE = 4 * EMB
VBLK = 4096


def _pack_block(i0, i1, i2, i3, o_ref):
    o_ref[...] = jnp.concatenate(
        [jnp.transpose(i0[...], (1, 0)), jnp.transpose(i1[...], (1, 0)),
         jnp.transpose(i2[...], (1, 0)), jnp.transpose(i3[...], (1, 0))],
        axis=1)


def _make_w_pack():
    last_blk = VOCAB // VBLK
    specs = [
        pl.BlockSpec((EMB, VBLK), functools.partial(
            lambda c, k: (0, jnp.minimum((VPAD // VBLK) * k + c, last_blk)),
            k=k))
        for k in range(4)
    ]
    return pl.pallas_call(
        _pack_block,
        grid=(VPAD // VBLK,),
        in_specs=specs,
        out_specs=pl.BlockSpec((VBLK, WIDE), lambda c: (c, 0)),
        out_shape=jax.ShapeDtypeStruct((VPAD, WIDE), jnp.float32),
    )


def _make_gather_kernel():
    mesh = plsc.VectorSubcoreMesh(core_axis_name="c", subcore_axis_name="s")

    @functools.partial(
        pl.kernel,
        out_type=(
            jax.ShapeDtypeStruct((NW, G, CHUNK, EMB), jnp.float32),
            jax.ShapeDtypeStruct((NW, G, CHUNK), jnp.float32),
        ),
        mesh=mesh,
        compiler_params=pltpu.CompilerParams(use_tc_tiling_on_sc=False),
        scratch_types=(
            [
                pltpu.VMEM((G, CHUNK), jnp.int32),
                pltpu.VMEM((RING, CHUNK), jnp.int32),
                pltpu.VMEM((RING, CHUNK, EMB), jnp.float32),
                pltpu.VMEM((G, CHUNK), jnp.float32),
            ]
            + [pltpu.SemaphoreType.DMA] * RING
            + [pltpu.SemaphoreType.DMA] * RING
        ),
    )
    def emb_kernel(x_hbm, w32_hbm, out_hbm, mask_hbm,
                   idx_v, row_v, rows_v, mask_v, *sems):
        gsems = sems[:RING]
        osems = sems[RING:]
        wid = lax.axis_index("s") * 2 + lax.axis_index("c")

        pltpu.sync_copy(x_hbm.at[wid], idx_v)

        def start_gather(g, r):
            for j in range(CHUNK // L):
                v = idx_v[g, pl.ds(j * L, L)]
                row_v[r, pl.ds(j * L, L)] = jnp.bitwise_or(
                    lax.shift_left(jnp.bitwise_and(v, VPAD - 1), 2),
                    lax.shift_right_logical(v, 18))
            pltpu.async_copy(
                w32_hbm.at[row_v.at[r]], rows_v.at[r], gsems[r])

        def wait_gather(r):
            pltpu.make_async_copy(
                w32_hbm.at[row_v.at[r]], rows_v.at[r], gsems[r]).wait()

        def start_out(g, r):
            pltpu.async_copy(rows_v.at[r], out_hbm.at[wid, g], osems[r])

        def wait_out(g, r):
            pltpu.make_async_copy(
                rows_v.at[r], out_hbm.at[wid, g], osems[r]).wait()

        def compute_mask(g):
            for j in range(CHUNK // L):
                v = idx_v[g, pl.ds(j * L, L)]
                mask_v[g, pl.ds(j * L, L)] = jnp.where(
                    v != 0, jnp.float32(1.0), jnp.float32(0.0))

        def visit(g, r, first_lap):
            compute_mask(g)
            wait_gather(r)
            start_out(g, r)
            g2 = g + AHEAD
            r2 = (r + AHEAD) % RING
            if first_lap:
                if g2 < RING:
                    start_gather(g2, r2)
                else:
                    wait_out(g2 - RING, r2)
                    start_gather(g2, r2)
            else:
                @pl.when(g2 < G)
                def _():
                    wait_out(g2 - RING, r2)
                    start_gather(g2, r2)

        for g in range(AHEAD):
            start_gather(g, g)
        for r in range(RING):
            visit(r, r, first_lap=True)

        def step(s, _):
            for r in range(RING):
                visit(s * RING + r, r, first_lap=False)
            return 0

        lax.fori_loop(1, G // RING, step, 0)

        for r in range(RING):
            wait_out(G - RING + r, r)
        pltpu.sync_copy(mask_v, mask_hbm.at[wid])

    return emb_kernel


_w_pack = None
_emb_kernel = None


def kernel(x, W):
    global _w_pack, _emb_kernel
    if _w_pack is None:
        _w_pack = _make_w_pack()
        _emb_kernel = _make_gather_kernel()
    wt = W.T
    w5 = _w_pack(wt, wt, wt, wt)
    w32 = w5.reshape(4 * VPAD, EMB)
    xf = x.reshape(NW, G, CHUNK).astype(jnp.int32)
    emb, mask = _emb_kernel(xf, w32)
    return emb.reshape(BATCH, SEQ, EMB), mask.reshape(BATCH, SEQ)


# speedup vs baseline: 3.8621x; 1.0053x over previous
import functools

import jax
import jax.numpy as jnp
from jax import lax
from jax.experimental import pallas as pl
from jax.experimental.pallas import tpu as pltpu
from jax.experimental.pallas import tpu_sc as plsc

VOCAB = 1000000
EMB = 32
BATCH = 4096
SEQ = 200
N = BATCH * SEQ
NW = 32
PER_W = N // NW
CHUNK = 128
G = PER_W // CHUNK
RING = 10
AHEAD = 5
L = 16

VPAD = 1 << 18
WIDE = 4 * EMB
VBLK = 8192


def _pack_block(i0, i1, i2, i3, o_ref):
    o_ref[...] = jnp.concatenate(
        [jnp.transpose(i0[...], (1, 0)), jnp.transpose(i1[...], (1, 0)),
         jnp.transpose(i2[...], (1, 0)), jnp.transpose(i3[...], (1, 0))],
        axis=1)


def _make_w_pack():
    last_blk = VOCAB // VBLK
    specs = [
        pl.BlockSpec((EMB, VBLK), functools.partial(
            lambda c, k: (0, jnp.minimum((VPAD // VBLK) * k + c, last_blk)),
            k=k))
        for k in range(4)
    ]
    return pl.pallas_call(
        _pack_block,
        grid=(VPAD // VBLK,),
        in_specs=specs,
        out_specs=pl.BlockSpec((VBLK, WIDE), lambda c: (c, 0)),
        out_shape=jax.ShapeDtypeStruct((VPAD, WIDE), jnp.float32),
    )


def _make_gather_kernel():
    mesh = plsc.VectorSubcoreMesh(core_axis_name="c", subcore_axis_name="s")

    @functools.partial(
        pl.kernel,
        out_type=(
            jax.ShapeDtypeStruct((NW, G, CHUNK, EMB), jnp.float32),
            jax.ShapeDtypeStruct((NW, G, CHUNK), jnp.float32),
        ),
        mesh=mesh,
        compiler_params=pltpu.CompilerParams(use_tc_tiling_on_sc=False),
        scratch_types=(
            [
                pltpu.VMEM((G, CHUNK), jnp.int32),
                pltpu.VMEM((RING, CHUNK), jnp.int32),
                pltpu.VMEM((RING, CHUNK, EMB), jnp.float32),
                pltpu.VMEM((G, CHUNK), jnp.float32),
            ]
            + [pltpu.SemaphoreType.DMA] * RING
            + [pltpu.SemaphoreType.DMA] * RING
        ),
    )
    def emb_kernel(x_hbm, w32_hbm, out_hbm, mask_hbm,
                   idx_v, row_v, rows_v, mask_v, *sems):
        gsems = sems[:RING]
        osems = sems[RING:]
        wid = lax.axis_index("s") * 2 + lax.axis_index("c")

        pltpu.sync_copy(x_hbm.at[wid], idx_v)

        def start_gather(g, r):
            for j in range(CHUNK // L):
                v = idx_v[g, pl.ds(j * L, L)]
                row_v[r, pl.ds(j * L, L)] = jnp.bitwise_or(
                    lax.shift_left(jnp.bitwise_and(v, VPAD - 1), 2),
                    lax.shift_right_logical(v, 18))
            pltpu.async_copy(
                w32_hbm.at[row_v.at[r]], rows_v.at[r], gsems[r])

        def wait_gather(r):
            pltpu.make_async_copy(
                w32_hbm.at[row_v.at[r]], rows_v.at[r], gsems[r]).wait()

        def start_out(g, r):
            pltpu.async_copy(rows_v.at[r], out_hbm.at[wid, g], osems[r])

        def wait_out(g, r):
            pltpu.make_async_copy(
                rows_v.at[r], out_hbm.at[wid, g], osems[r]).wait()

        def compute_mask(g):
            for j in range(CHUNK // L):
                v = idx_v[g, pl.ds(j * L, L)]
                mask_v[g, pl.ds(j * L, L)] = jnp.where(
                    v != 0, jnp.float32(1.0), jnp.float32(0.0))

        def visit(g, r, first_lap):
            compute_mask(g)
            wait_gather(r)
            start_out(g, r)
            g2 = g + AHEAD
            r2 = (r + AHEAD) % RING
            if first_lap:
                if g2 < RING:
                    start_gather(g2, r2)
                else:
                    wait_out(g2 - RING, r2)
                    start_gather(g2, r2)
            else:
                @pl.when(g2 < G)
                def _():
                    wait_out(g2 - RING, r2)
                    start_gather(g2, r2)

        for g in range(AHEAD):
            start_gather(g, g)
        for r in range(RING):
            visit(r, r, first_lap=True)

        def step(s, _):
            for r in range(RING):
                visit(s * RING + r, r, first_lap=False)
            return 0

        lax.fori_loop(1, G // RING, step, 0)

        for r in range(RING):
            wait_out(G - RING + r, r)
        pltpu.sync_copy(mask_v, mask_hbm.at[wid])

    return emb_kernel


_w_pack = None
_emb_kernel = None


def kernel(x, W):
    global _w_pack, _emb_kernel
    if _w_pack is None:
        _w_pack = _make_w_pack()
        _emb_kernel = _make_gather_kernel()
    wt = W.T
    w5 = _w_pack(wt, wt, wt, wt)
    w32 = w5.reshape(4 * VPAD, EMB)
    xf = x.reshape(NW, G, CHUNK).astype(jnp.int32)
    emb, mask = _emb_kernel(xf, w32)
    return emb.reshape(BATCH, SEQ, EMB), mask.reshape(BATCH, SEQ)
